# Initial kernel scaffold; baseline (speedup 1.0000x reference)
import jax
import jax.numpy as jnp
from jax.experimental import pallas as pl


def kernel(center_nodes, context_nodes, embeddings, output_embeddings):
    raise NotImplementedError("write your pallas kernel here")


# speedup vs baseline: 1.1522x; 1.1522x over previous
import functools

import numpy as np
import jax
import jax.numpy as jnp
from jax import lax
from jax.experimental import pallas as pl
from jax.experimental.pallas import tpu as pltpu
from jax.experimental.pallas import tpu_sc as plsc

B = 16384
D = 128

_info = plsc.get_sparse_core_info()
NC = _info.num_cores
NS = _info.num_subcores
L = _info.num_lanes
NW = NC * NS
BPW = B // NW
CHUNK = 128
NCHUNK = BPW // CHUNK
GROUPS = CHUNK // L

_BFLY_SHIFTS = (8, 4, 2, 1)


def _skipgram_body(center_hbm, context_hbm, emb_hbm, oemb_hbm, out_hbm,
                   idx_c, idx_x, rc0, rc1, rx0, rx1, scores,
                   s0, s1, s2, s3):
    wid = lax.axis_index("s") * NC + lax.axis_index("c")
    base = wid * BPW

    pltpu.sync_copy(center_hbm.at[pl.ds(base, BPW)], idx_c)
    pltpu.sync_copy(context_hbm.at[pl.ds(base, BPW)], idx_x)

    rc = (rc0, rc1)
    rx = (rx0, rx1)
    sem_c = (s0, s1)
    sem_x = (s2, s3)

    def start(c):
        b = c & 1
        hc = pltpu.async_copy(
            emb_hbm.at[idx_c.at[pl.ds(c * CHUNK, CHUNK)]], rc[b], sem_c[b])
        hx = pltpu.async_copy(
            oemb_hbm.at[idx_x.at[pl.ds(c * CHUNK, CHUNK)]], rx[b], sem_x[b])
        return hc, hx

    pending = start(0)
    for c in range(NCHUNK):
        b = c & 1
        nxt = start(c + 1) if c + 1 < NCHUNK else None
        pending[0].wait()
        pending[1].wait()
        rcb = rc[b]
        rxb = rx[b]

        def group(g, carry, rcb=rcb, rxb=rxb, c=c):
            lane = lax.iota(jnp.int32, L)
            perms = [lane ^ s for s in _BFLY_SHIFTS]
            vec = None
            row0 = pl.multiple_of(g * L, L)
            for i in range(L):
                r = row0 + i
                acc = rcb[r, pl.ds(0, L)] * rxb[r, pl.ds(0, L)]
                for d in range(1, D // L):
                    acc = acc + rcb[r, pl.ds(d * L, L)] * rxb[r, pl.ds(d * L, L)]
                for perm in perms:
                    acc = acc + acc.at[perm].get(mode="promise_in_bounds")
                hit = lane == i
                vec = jnp.where(hit, acc, 0.0) if vec is None else jnp.where(hit, acc, vec)
            scores[pl.ds(c * CHUNK + row0, L)] = vec
            return carry

        lax.fori_loop(0, GROUPS, group, 0)
        pending = nxt

    pltpu.sync_copy(scores, out_hbm.at[pl.ds(base, BPW)])


_skipgram = functools.partial(
    pl.kernel,
    out_type=jax.ShapeDtypeStruct((B,), jnp.float32),
    mesh=plsc.VectorSubcoreMesh(core_axis_name="c", subcore_axis_name="s"),
    scratch_types=[
        pltpu.VMEM((BPW,), jnp.int32),
        pltpu.VMEM((BPW,), jnp.int32),
        pltpu.VMEM((CHUNK, D), jnp.float32),
        pltpu.VMEM((CHUNK, D), jnp.float32),
        pltpu.VMEM((CHUNK, D), jnp.float32),
        pltpu.VMEM((CHUNK, D), jnp.float32),
        pltpu.VMEM((BPW,), jnp.float32),
        pltpu.SemaphoreType.DMA,
        pltpu.SemaphoreType.DMA,
        pltpu.SemaphoreType.DMA,
        pltpu.SemaphoreType.DMA,
    ],
)(_skipgram_body)


def kernel(center_nodes, context_nodes, embeddings, output_embeddings):
    return _skipgram(center_nodes, context_nodes, embeddings,
                     output_embeddings)
